# baseline (speedup 1.0000x reference)
import functools

import jax
import jax.numpy as jnp
from jax import lax
from jax.experimental import pallas as pl
from jax.experimental.pallas import tpu as pltpu
from jax.experimental.pallas import tpu_sc as plsc

NUM_FIELDS = 26
VOCAB = 100000
EMBED_DIM = 32
BATCH = 4096
SEQ = 20

NC = 2
NS = 16
NW = NC * NS

OUT_BYTES = BATCH * 4


def _make_lookup():
    mesh = plsc.VectorSubcoreMesh(core_axis_name="c", subcore_axis_name="s")

    @functools.partial(
        pl.kernel,
        mesh=mesh,
        out_type=jax.ShapeDtypeStruct((SEQ, NUM_FIELDS * EMBED_DIM, BATCH),
                                      jnp.float32),
        scratch_types=[
            pltpu.VMEM((VOCAB,), jnp.float32),
            pltpu.VMEM((BATCH,), jnp.int32),
            pltpu.VMEM((BATCH,), jnp.int32),
            pltpu.VMEM((BATCH,), jnp.float32),
            pltpu.VMEM((BATCH,), jnp.float32),
            pltpu.VMEM_SHARED((2, 24, BATCH), jnp.int32),
            pltpu.SemaphoreType.DMA,
            pltpu.SemaphoreType.DMA,
            pltpu.SemaphoreType.DMA,
            pltpu.SemaphoreType.DMA,
            pltpu.SemaphoreType.DMA,
            pltpu.SemaphoreType.DMA,
        ],
        compiler_params=pltpu.CompilerParams(needs_layout_passes=False),
    )
    def lookup_kernel(tabT_hbm, xT_hbm, out_hbm, row_v,
                      idx0, idx1, out0, out1, xsh,
                      si0, si1, sw0, sw1, sxh, sr):
        cid = lax.axis_index("c")
        sid = lax.axis_index("s")
        d = sid * NC + cid
        idx_b = (idx0, idx1)
        out_b = (out0, out1)
        si = (si0, si1)
        sw = (sw0, sw1)

        pltpu.async_copy(out0, out_hbm.at[0, d], sw0)
        pltpu.async_copy(out1, out_hbm.at[1, d], sw1)

        def step(f, slot, s, b):
            pltpu.make_async_copy(xsh.at[slot, s], idx_b[b], si[b]).wait()
            pltpu.make_async_copy(
                out_b[b], out_hbm.at[s, f * EMBED_DIM + d], sw[b]).wait()

            for j in range(BATCH // 16):
                sl = pl.ds(j * 16, 16)
                out_b[b][sl] = plsc.load_gather(row_v, [idx_b[b][sl]])

            pltpu.async_copy(out_b[b], out_hbm.at[s, f * EMBED_DIM + d], sw[b])

            @pl.when(s + 2 < SEQ)
            def _():
                pltpu.async_copy(xsh.at[slot, s + 2], idx_b[b], si[b])

        def field(f, carry):
            slot = f % 2
            pltpu.async_copy(tabT_hbm.at[f, d], row_v, sr)

            @pl.when(sid == 0)
            def _():
                for s in range(SEQ):
                    pltpu.make_async_copy(
                        xT_hbm.at[f, s], xsh.at[slot, s], sxh).wait()

            plsc.subcore_barrier()

            @pl.when(jnp.logical_and(sid == 0, f + 1 < NUM_FIELDS))
            def _():
                for s in range(SEQ):
                    pltpu.async_copy(
                        xT_hbm.at[f + 1, s], xsh.at[(f + 1) % 2, s], sxh)

            pltpu.async_copy(xsh.at[slot, 0], idx0, si0)
            pltpu.async_copy(xsh.at[slot, 1], idx1, si1)
            pltpu.make_async_copy(tabT_hbm.at[f, d], row_v, sr).wait()

            def spair_body(q, carry2):
                step(f, slot, 2 * q, 0)
                step(f, slot, 2 * q + 1, 1)
                return carry2

            lax.fori_loop(0, SEQ // 2, spair_body, 0)
            return carry

        @pl.when(sid == 0)
        def _():
            for s in range(SEQ):
                pltpu.async_copy(xT_hbm.at[0, s], xsh.at[0, s], sxh)

        lax.fori_loop(0, NUM_FIELDS, field, 0)

        pltpu.make_async_copy(out0, out_hbm.at[0, d], sw0).wait()
        pltpu.make_async_copy(out1, out_hbm.at[0, d], sw1).wait()

    return lookup_kernel


_lookup = _make_lookup()


def kernel(x, tables):
    tabT = jnp.transpose(tables, (0, 2, 1))
    xT = jnp.transpose(x, (2, 1, 0))
    out3 = _lookup(tabT, xT)
    return jnp.transpose(out3, (2, 0, 1))


# speedup vs baseline: 2.3873x; 2.3873x over previous
import functools

import jax
import jax.numpy as jnp
from jax import lax
from jax.experimental import pallas as pl
from jax.experimental.pallas import tpu as pltpu
from jax.experimental.pallas import tpu_sc as plsc

NUM_FIELDS = 26
VOCAB = 100000
EMBED_DIM = 32
BATCH = 4096
SEQ = 20

NC = 2
NS = 16
NW = NC * NS

OUT_BYTES = BATCH * 4


def _make_lookup():
    mesh = plsc.VectorSubcoreMesh(core_axis_name="c", subcore_axis_name="s")

    @functools.partial(
        pl.kernel,
        mesh=mesh,
        out_type=jax.ShapeDtypeStruct((SEQ, NUM_FIELDS * EMBED_DIM, BATCH),
                                      jnp.float32),
        scratch_types=[
            pltpu.VMEM((VOCAB,), jnp.float32),
            pltpu.VMEM((BATCH,), jnp.int32),
            pltpu.VMEM((BATCH,), jnp.int32),
            pltpu.VMEM((BATCH,), jnp.float32),
            pltpu.VMEM((BATCH,), jnp.float32),
            pltpu.VMEM_SHARED((2, 24, BATCH), jnp.int32),
            pltpu.SemaphoreType.DMA,
            pltpu.SemaphoreType.DMA,
            pltpu.SemaphoreType.DMA,
            pltpu.SemaphoreType.DMA,
            pltpu.SemaphoreType.DMA,
            pltpu.SemaphoreType.DMA,
        ],
        compiler_params=pltpu.CompilerParams(needs_layout_passes=False),
    )
    def lookup_kernel(tabT_hbm, xT_hbm, out_hbm, row_v,
                      idx0, idx1, out0, out1, xsh,
                      si0, si1, sw0, sw1, sxh, sr):
        cid = lax.axis_index("c")
        sid = lax.axis_index("s")
        d = sid * NC + cid
        idx_b = (idx0, idx1)
        out_b = (out0, out1)
        si = (si0, si1)
        sw = (sw0, sw1)

        pltpu.async_copy(out0, out_hbm.at[0, d], sw0)
        pltpu.async_copy(out1, out_hbm.at[1, d], sw1)

        def step(f, slot, s, b):
            pltpu.make_async_copy(xsh.at[slot, s], idx_b[b], si[b]).wait()
            pltpu.make_async_copy(
                out_b[b], out_hbm.at[s, f * EMBED_DIM + d], sw[b]).wait()

            @plsc.parallel_loop(0, BATCH // 16, unroll=8)
            def _(j):
                sl = pl.ds(j * 16, 16)
                out_b[b][sl] = plsc.load_gather(row_v, [idx_b[b][sl]])

            pltpu.async_copy(out_b[b], out_hbm.at[s, f * EMBED_DIM + d], sw[b])

            @pl.when(s + 2 < SEQ)
            def _():
                pltpu.async_copy(xsh.at[slot, s + 2], idx_b[b], si[b])

        def field(f, carry):
            slot = f % 2
            pltpu.async_copy(tabT_hbm.at[f, d], row_v, sr)

            @pl.when(sid == 0)
            def _():
                for s in range(SEQ):
                    pltpu.make_async_copy(
                        xT_hbm.at[f, s], xsh.at[slot, s], sxh).wait()

            plsc.subcore_barrier()

            @pl.when(jnp.logical_and(sid == 0, f + 1 < NUM_FIELDS))
            def _():
                for s in range(SEQ):
                    pltpu.async_copy(
                        xT_hbm.at[f + 1, s], xsh.at[(f + 1) % 2, s], sxh)

            pltpu.async_copy(xsh.at[slot, 0], idx0, si0)
            pltpu.async_copy(xsh.at[slot, 1], idx1, si1)
            pltpu.make_async_copy(tabT_hbm.at[f, d], row_v, sr).wait()

            def spair_body(q, carry2):
                step(f, slot, 2 * q, 0)
                step(f, slot, 2 * q + 1, 1)
                return carry2

            lax.fori_loop(0, SEQ // 2, spair_body, 0)
            return carry

        @pl.when(sid == 0)
        def _():
            for s in range(SEQ):
                pltpu.async_copy(xT_hbm.at[0, s], xsh.at[0, s], sxh)

        lax.fori_loop(0, NUM_FIELDS, field, 0)

        pltpu.make_async_copy(out0, out_hbm.at[0, d], sw0).wait()
        pltpu.make_async_copy(out1, out_hbm.at[0, d], sw1).wait()

    return lookup_kernel


_lookup = _make_lookup()


def kernel(x, tables):
    tabT = jnp.transpose(tables, (0, 2, 1))
    xT = jnp.transpose(x, (2, 1, 0))
    out3 = _lookup(tabT, xT)
    return jnp.transpose(out3, (2, 0, 1))


# speedup vs baseline: 2.3942x; 1.0029x over previous
import functools

import jax
import jax.numpy as jnp
from jax import lax
from jax.experimental import pallas as pl
from jax.experimental.pallas import tpu as pltpu
from jax.experimental.pallas import tpu_sc as plsc

NUM_FIELDS = 26
VOCAB = 100000
EMBED_DIM = 32
BATCH = 4096
SEQ = 20

NC = 2
NS = 16
NW = NC * NS

OUT_BYTES = BATCH * 4


def _make_lookup():
    mesh = plsc.VectorSubcoreMesh(core_axis_name="c", subcore_axis_name="s")

    @functools.partial(
        pl.kernel,
        mesh=mesh,
        out_type=jax.ShapeDtypeStruct((SEQ, NUM_FIELDS * EMBED_DIM, BATCH),
                                      jnp.float32),
        scratch_types=[
            pltpu.VMEM((VOCAB,), jnp.float32),
            pltpu.VMEM((BATCH,), jnp.int32),
            pltpu.VMEM((BATCH,), jnp.int32),
            pltpu.VMEM((BATCH,), jnp.float32),
            pltpu.VMEM((BATCH,), jnp.float32),
            pltpu.VMEM_SHARED((2, 24, BATCH), jnp.int32),
            pltpu.SemaphoreType.DMA,
            pltpu.SemaphoreType.DMA,
            pltpu.SemaphoreType.DMA,
            pltpu.SemaphoreType.DMA,
            pltpu.SemaphoreType.DMA,
            pltpu.SemaphoreType.DMA,
        ],
        compiler_params=pltpu.CompilerParams(needs_layout_passes=False),
    )
    def lookup_kernel(tabT_hbm, xT_hbm, out_hbm, row_v,
                      idx0, idx1, out0, out1, xsh,
                      si0, si1, sw0, sw1, sxh, sr):
        cid = lax.axis_index("c")
        sid = lax.axis_index("s")
        d = sid * NC + cid
        idx_b = (idx0, idx1)
        out_b = (out0, out1)
        si = (si0, si1)
        sw = (sw0, sw1)

        pltpu.async_copy(out0, out_hbm.at[0, d], sw0)
        pltpu.async_copy(out1, out_hbm.at[1, d], sw1)

        def step(f, slot, s, b):
            pltpu.make_async_copy(xsh.at[slot, s], idx_b[b], si[b]).wait()
            pltpu.make_async_copy(
                out_b[b], out_hbm.at[s, f * EMBED_DIM + d], sw[b]).wait()

            @plsc.parallel_loop(0, BATCH // 16, unroll=16)
            def _(j):
                sl = pl.ds(j * 16, 16)
                out_b[b][sl] = plsc.load_gather(row_v, [idx_b[b][sl]])

            pltpu.async_copy(out_b[b], out_hbm.at[s, f * EMBED_DIM + d], sw[b])

            @pl.when(s + 2 < SEQ)
            def _():
                pltpu.async_copy(xsh.at[slot, s + 2], idx_b[b], si[b])

        def field(f, carry):
            slot = f % 2
            pltpu.async_copy(tabT_hbm.at[f, d], row_v, sr)

            @pl.when(sid == 0)
            def _():
                for s in range(SEQ):
                    pltpu.make_async_copy(
                        xT_hbm.at[f, s], xsh.at[slot, s], sxh).wait()

            plsc.subcore_barrier()

            @pl.when(jnp.logical_and(sid == 0, f + 1 < NUM_FIELDS))
            def _():
                for s in range(SEQ):
                    pltpu.async_copy(
                        xT_hbm.at[f + 1, s], xsh.at[(f + 1) % 2, s], sxh)

            pltpu.async_copy(xsh.at[slot, 0], idx0, si0)
            pltpu.async_copy(xsh.at[slot, 1], idx1, si1)
            pltpu.make_async_copy(tabT_hbm.at[f, d], row_v, sr).wait()

            def spair_body(q, carry2):
                step(f, slot, 2 * q, 0)
                step(f, slot, 2 * q + 1, 1)
                return carry2

            lax.fori_loop(0, SEQ // 2, spair_body, 0)
            return carry

        @pl.when(sid == 0)
        def _():
            for s in range(SEQ):
                pltpu.async_copy(xT_hbm.at[0, s], xsh.at[0, s], sxh)

        lax.fori_loop(0, NUM_FIELDS, field, 0)

        pltpu.make_async_copy(out0, out_hbm.at[0, d], sw0).wait()
        pltpu.make_async_copy(out1, out_hbm.at[0, d], sw1).wait()

    return lookup_kernel


_lookup = _make_lookup()


def kernel(x, tables):
    tabT = jnp.transpose(tables, (0, 2, 1))
    xT = jnp.transpose(x, (2, 1, 0))
    out3 = _lookup(tabT, xT)
    return jnp.transpose(out3, (2, 0, 1))
